# Initial kernel scaffold; baseline (speedup 1.0000x reference)
import functools

import jax
import jax.numpy as jnp
from jax import lax
from jax.experimental import pallas as pl
from jax.experimental.pallas import tpu as pltpu
from jax.experimental.pallas import tpu_sc as plsc

N_CORES = 2
N_SUBCORES = 16
N_WORKERS = N_CORES * N_SUBCORES
CHUNK = 80
F32 = jnp.float32


def _zero_shared(zbuf, acc_sh, sid, rows_per_tile, tiles):
    n_copies = rows_per_tile // 125

    @pl.when(sid < tiles)
    def _():
        def body(j, carry):
            pltpu.sync_copy(zbuf, acc_sh.at[pl.ds(sid * rows_per_tile + j * 125, 125)])
            return carry
        lax.fori_loop(0, n_copies, body, 0)


def _fill_const(buf, rows, feat, value):
    per_row = feat // 16
    vec = jnp.full((16,), value, F32)

    def body(t, carry):
        i = t // per_row
        j = t % per_row
        buf[i, pl.ds(j * 16, 16)] = vec
        return carry
    lax.fori_loop(0, rows * per_row, body, 0)


def _sc_counts(edge_index, pool1, n_nodes, n_edges, n_pool):
    per_worker = n_edges // N_WORKERS
    n_chunks = per_worker // CHUNK
    pool_chunks = n_nodes // CHUNK
    pool_iters = (pool_chunks + N_WORKERS - 1) // N_WORKERS
    mesh = plsc.VectorSubcoreMesh(core_axis_name="c", subcore_axis_name="s")

    @functools.partial(
        pl.kernel,
        out_type=(
            jax.ShapeDtypeStruct((N_CORES, n_nodes, 16), F32),
            jax.ShapeDtypeStruct((N_CORES, n_pool, 16), F32),
        ),
        mesh=mesh,
        scratch_types=[
            pltpu.VMEM((CHUNK,), jnp.int32),
            pltpu.VMEM((CHUNK, 16), F32),
            pltpu.VMEM((125, 16), F32),
            pltpu.VMEM_SHARED((n_nodes, 16), F32),
            pltpu.VMEM_SHARED((n_pool, 16), F32),
        ],
    )
    def counts(edge_hbm, pool_hbm, deg_hbm, cnt_hbm, idx_v, ones_v, zbuf,
               deg_sh, cnt_sh):
        c = lax.axis_index("c")
        s = lax.axis_index("s")
        w = s * N_CORES + c

        _fill_const(zbuf, 125, 16, 0.0)
        _fill_const(ones_v, CHUNK, 16, 1.0)
        _zero_shared(zbuf, deg_sh, s, n_nodes // N_SUBCORES, N_SUBCORES)
        _zero_shared(zbuf, cnt_sh, s, 125, n_pool // 125)
        plsc.subcore_barrier()

        base0 = w * per_worker

        def ebody(i, carry):
            pltpu.sync_copy(edge_hbm.at[1, pl.ds(base0 + i * CHUNK, CHUNK)], idx_v)
            pltpu.sync_copy(ones_v, deg_sh.at[idx_v], add=True)
            return carry
        lax.fori_loop(0, n_chunks, ebody, 0)

        def pbody(i, carry):
            cidx = w + i * N_WORKERS

            @pl.when(cidx < pool_chunks)
            def _():
                pltpu.sync_copy(pool_hbm.at[pl.ds(cidx * CHUNK, CHUNK)], idx_v)
                pltpu.sync_copy(ones_v, cnt_sh.at[idx_v], add=True)
            return carry
        lax.fori_loop(0, pool_iters, pbody, 0)

        plsc.subcore_barrier()
        rpt = n_nodes // N_SUBCORES
        pltpu.sync_copy(deg_sh.at[pl.ds(s * rpt, rpt)],
                        deg_hbm.at[c, pl.ds(s * rpt, rpt)])

        @pl.when(s < n_pool // 125)
        def _():
            pltpu.sync_copy(cnt_sh.at[pl.ds(s * 125, 125)],
                            cnt_hbm.at[c, pl.ds(s * 125, 125)])

    return counts(edge_index, pool1)


def _sc_message(msgs, edge_index, n_nodes, n_edges, feat):
    per_worker = n_edges // N_WORKERS
    n_chunks = per_worker // CHUNK
    rpt = n_nodes // N_SUBCORES
    mesh = plsc.VectorSubcoreMesh(core_axis_name="c", subcore_axis_name="s")

    @functools.partial(
        pl.kernel,
        out_type=jax.ShapeDtypeStruct((N_CORES, n_nodes, feat), F32),
        mesh=mesh,
        scratch_types=[
            pltpu.VMEM((CHUNK,), jnp.int32),
            pltpu.VMEM((CHUNK,), jnp.int32),
            pltpu.VMEM((CHUNK, feat), F32),
            pltpu.VMEM((125, feat), F32),
            pltpu.VMEM_SHARED((n_nodes, feat), F32),
            pltpu.SemaphoreType.DMA,
        ],
    )
    def msg(msgs_hbm, edge_hbm, out_hbm, src_v, dst_v, rows_v, zbuf, acc_sh,
            sem):
        c = lax.axis_index("c")
        s = lax.axis_index("s")
        w = s * N_CORES + c

        _fill_const(zbuf, 125, feat, 0.0)
        _zero_shared(zbuf, acc_sh, s, rpt, N_SUBCORES)
        plsc.subcore_barrier()

        base0 = w * per_worker

        def ebody(i, carry):
            base = base0 + i * CHUNK
            pltpu.sync_copy(edge_hbm.at[0, pl.ds(base, CHUNK)], src_v)
            pltpu.sync_copy(edge_hbm.at[1, pl.ds(base, CHUNK)], dst_v)
            pltpu.async_copy(msgs_hbm.at[src_v], rows_v, sem).wait()
            pltpu.sync_copy(rows_v, acc_sh.at[dst_v], add=True)
            return carry
        lax.fori_loop(0, n_chunks, ebody, 0)

        plsc.subcore_barrier()
        pltpu.sync_copy(acc_sh.at[pl.ds(s * rpt, rpt)],
                        out_hbm.at[c, pl.ds(s * rpt, rpt)])

    return msg(msgs, edge_index)


def _sc_pool(x_mid, pool1, n_nodes, n_pool, feat):
    pool_chunks = n_nodes // CHUNK
    pool_iters = (pool_chunks + N_WORKERS - 1) // N_WORKERS
    mesh = plsc.VectorSubcoreMesh(core_axis_name="c", subcore_axis_name="s")

    @functools.partial(
        pl.kernel,
        out_type=jax.ShapeDtypeStruct((N_CORES, n_pool, feat), F32),
        mesh=mesh,
        scratch_types=[
            pltpu.VMEM((CHUNK,), jnp.int32),
            pltpu.VMEM((CHUNK, feat), F32),
            pltpu.VMEM((125, feat), F32),
            pltpu.VMEM_SHARED((n_pool, feat), F32),
        ],
    )
    def pool(x_hbm, pool_hbm, out_hbm, idx_v, rows_v, zbuf, acc_sh):
        c = lax.axis_index("c")
        s = lax.axis_index("s")
        w = s * N_CORES + c

        _fill_const(zbuf, 125, feat, 0.0)
        _zero_shared(zbuf, acc_sh, s, 125, n_pool // 125)
        plsc.subcore_barrier()

        def pbody(i, carry):
            cidx = w + i * N_WORKERS

            @pl.when(cidx < pool_chunks)
            def _():
                pltpu.sync_copy(pool_hbm.at[pl.ds(cidx * CHUNK, CHUNK)], idx_v)
                pltpu.sync_copy(x_hbm.at[pl.ds(cidx * CHUNK, CHUNK)], rows_v)
                pltpu.sync_copy(rows_v, acc_sh.at[idx_v], add=True)
            return carry
        lax.fori_loop(0, pool_iters, pbody, 0)

        plsc.subcore_barrier()

        @pl.when(s < n_pool // 125)
        def _():
            pltpu.sync_copy(acc_sh.at[pl.ds(s * 125, 125)],
                            out_hbm.at[c, pl.ds(s * 125, 125)])

    return pool(x_mid, pool1)


_BLK = 1000


def _tc_prep0(x, W0, degp):
    n, d_in = x.shape
    d_out = W0.shape[1]

    def body(x_ref, w_ref, degp_ref, m_ref, dis_ref):
        deg = degp_ref[0, :, 0] + degp_ref[1, :, 0]
        dis = jnp.where(deg > 0.0, lax.rsqrt(jnp.maximum(deg, 1.0)), 0.0)
        h = jnp.dot(x_ref[...], w_ref[...], preferred_element_type=F32)
        m_ref[...] = h * dis[:, None]
        dis_ref[...] = dis[:, None]

    return pl.pallas_call(
        body,
        grid=(n // _BLK,),
        in_specs=[
            pl.BlockSpec((_BLK, d_in), lambda i: (i, 0)),
            pl.BlockSpec((d_in, d_out), lambda i: (0, 0)),
            pl.BlockSpec((2, _BLK, 16), lambda i: (0, i, 0)),
        ],
        out_specs=[
            pl.BlockSpec((_BLK, d_out), lambda i: (i, 0)),
            pl.BlockSpec((_BLK, 1), lambda i: (i, 0)),
        ],
        out_shape=[
            jax.ShapeDtypeStruct((n, d_out), F32),
            jax.ShapeDtypeStruct((n, 1), F32),
        ],
    )(x, W0, degp)


def _tc_prep_mid(partials, dis, b_prev, W):
    n = partials.shape[1]
    f_prev = partials.shape[2]
    f_out = W.shape[1]
    b2d = b_prev.reshape(1, f_prev)

    def body(p_ref, dis_ref, b_ref, w_ref, m_ref):
        ssum = p_ref[0] + p_ref[1]
        a = jnp.maximum(ssum * dis_ref[...] + b_ref[...], 0.0)
        m_ref[...] = jnp.dot(a, w_ref[...], preferred_element_type=F32) * dis_ref[...]

    return pl.pallas_call(
        body,
        grid=(n // _BLK,),
        in_specs=[
            pl.BlockSpec((2, _BLK, f_prev), lambda i: (0, i, 0)),
            pl.BlockSpec((_BLK, 1), lambda i: (i, 0)),
            pl.BlockSpec((1, f_prev), lambda i: (0, 0)),
            pl.BlockSpec((f_prev, f_out), lambda i: (0, 0)),
        ],
        out_specs=pl.BlockSpec((_BLK, f_out), lambda i: (i, 0)),
        out_shape=jax.ShapeDtypeStruct((n, f_out), F32),
    )(partials, dis, b2d, W)


def _tc_post2(partials, dis, b2):
    n = partials.shape[1]
    f = partials.shape[2]
    b2d = b2.reshape(1, f)

    def body(p_ref, dis_ref, b_ref, o_ref):
        o_ref[...] = (p_ref[0] + p_ref[1]) * dis_ref[...] + b_ref[...]

    return pl.pallas_call(
        body,
        grid=(n // _BLK,),
        in_specs=[
            pl.BlockSpec((2, _BLK, f), lambda i: (0, i, 0)),
            pl.BlockSpec((_BLK, 1), lambda i: (i, 0)),
            pl.BlockSpec((1, f), lambda i: (0, 0)),
        ],
        out_specs=pl.BlockSpec((_BLK, f), lambda i: (i, 0)),
        out_shape=jax.ShapeDtypeStruct((n, f), F32),
    )(partials, dis, b2d)


def _tc_final(poolp, cntp, x_pool1, W_lin, b_lin):
    n_pool = poolp.shape[1]
    f = poolp.shape[2]
    n_classes = W_lin.shape[1]
    b2d = b_lin.reshape(1, n_classes)

    def body(q_ref, c_ref, xp_ref, wa_ref, wb_ref, b_ref, o_ref):
        q = q_ref[0] + q_ref[1]
        cnt = c_ref[0, :, 0] + c_ref[1, :, 0]
        x_pre = q / jnp.maximum(cnt, 1.0)[:, None]
        m_a = jnp.sum(x_pre, axis=0, keepdims=True) / n_pool
        m_b = jnp.sum(xp_ref[...], axis=0, keepdims=True) / n_pool
        logits = (jnp.dot(m_a, wa_ref[...], preferred_element_type=F32)
                  + jnp.dot(m_b, wb_ref[...], preferred_element_type=F32)
                  + b_ref[...])
        e = jnp.exp(logits - jnp.max(logits, axis=1, keepdims=True))
        o_ref[...] = e / jnp.sum(e, axis=1, keepdims=True)

    return pl.pallas_call(
        body,
        out_shape=jax.ShapeDtypeStruct((1, n_classes), F32),
    )(poolp, cntp, x_pool1, W_lin[:f], W_lin[f:], b2d)


def kernel(x, edge_index, batch, pool1, x_pool1, W0, b0, W1, b1, W2, b2,
           W_lin, b_lin):
    n_nodes = x.shape[0]
    n_edges = edge_index.shape[1]
    n_pool = x_pool1.shape[0]

    degp, cntp = _sc_counts(edge_index, pool1, n_nodes, n_edges, n_pool)

    m0, dis = _tc_prep0(x, W0, degp)
    p0 = _sc_message(m0, edge_index, n_nodes, n_edges, W0.shape[1])
    m1 = _tc_prep_mid(p0, dis, b0, W1)
    p1 = _sc_message(m1, edge_index, n_nodes, n_edges, W1.shape[1])
    m2 = _tc_prep_mid(p1, dis, b1, W2)
    p2 = _sc_message(m2, edge_index, n_nodes, n_edges, W2.shape[1])
    x_mid = _tc_post2(p2, dis, b2)

    poolp = _sc_pool(x_mid, pool1, n_nodes, n_pool, x_mid.shape[1])
    return _tc_final(poolp, cntp, x_pool1, W_lin, b_lin)


# speedup vs baseline: 29.5388x; 29.5388x over previous
import functools

import jax
import jax.numpy as jnp
from jax import lax
from jax.experimental import pallas as pl
from jax.experimental.pallas import tpu as pltpu
from jax.experimental.pallas import tpu_sc as plsc

N_CORES = 2
N_SUBCORES = 16
N_WORKERS = N_CORES * N_SUBCORES
CHUNK = 80
F32 = jnp.float32
BF16 = jnp.bfloat16


STRIPE = 200


def _for_stripes(sid, n_rows, fn):
    n_stripes = n_rows // STRIPE
    n_iters = (n_stripes + N_SUBCORES - 1) // N_SUBCORES

    def body(i, carry):
        k = sid + i * N_SUBCORES

        @pl.when(k < n_stripes)
        def _():
            fn(k)
        return carry
    lax.fori_loop(0, n_iters, body, 0)


def _zero_shared(zbuf, acc_sh, sid, n_rows, stripe=STRIPE):
    n_stripes = n_rows // stripe
    n_iters = (n_stripes + N_SUBCORES - 1) // N_SUBCORES

    def body(i, carry):
        k = sid + i * N_SUBCORES

        @pl.when(k < n_stripes)
        def _():
            pltpu.sync_copy(zbuf, acc_sh.at[pl.ds(k * stripe, stripe)])
        return carry
    lax.fori_loop(0, n_iters, body, 0)


def _copy_out(acc_sh, out_hbm, cid, sid, n_rows):
    _for_stripes(sid, n_rows,
                 lambda k: pltpu.sync_copy(acc_sh.at[pl.ds(k * STRIPE, STRIPE)],
                                           out_hbm.at[cid, pl.ds(k * STRIPE, STRIPE)]))


def _fill_const(buf, rows, feat, value, dtype=F32):
    lanes = 32 if dtype == BF16 else 16
    per_row = feat // lanes
    vec = jnp.full((lanes,), value, dtype)

    def body(t, carry):
        i = t // per_row
        j = t % per_row
        buf[i, pl.ds(j * lanes, lanes)] = vec
        return carry
    lax.fori_loop(0, rows * per_row, body, 0)


def _sc_counts(edge3d, pool2d, n_nodes, n_edges, n_pool):
    per_worker = n_edges // N_WORKERS
    rows_pw = per_worker // CHUNK
    pool_rows = n_nodes // CHUNK
    pool_iters = (pool_rows + N_WORKERS - 1) // N_WORKERS
    mesh = plsc.VectorSubcoreMesh(core_axis_name="c", subcore_axis_name="s")

    @functools.partial(
        pl.kernel,
        out_type=(
            jax.ShapeDtypeStruct((N_CORES, n_nodes, 16), F32),
            jax.ShapeDtypeStruct((N_CORES, n_pool, 16), F32),
        ),
        mesh=mesh,
        compiler_params=pltpu.CompilerParams(use_tc_tiling_on_sc=False),
        scratch_types=[
            pltpu.VMEM((rows_pw, CHUNK), jnp.int32),
            pltpu.VMEM((CHUNK,), jnp.int32),
            pltpu.VMEM((CHUNK, 16), F32),
            pltpu.VMEM((STRIPE, 16), F32),
            pltpu.VMEM_SHARED((n_nodes, 16), F32),
            pltpu.VMEM_SHARED((n_pool, 16), F32),
        ],
    )
    def counts(edge_hbm, pool_hbm, deg_hbm, cnt_hbm, dst_all, idx_v, ones_v,
               zbuf, deg_sh, cnt_sh):
        c = lax.axis_index("c")
        s = lax.axis_index("s")
        w = s * N_CORES + c

        pltpu.sync_copy(edge_hbm.at[1, pl.ds(w * rows_pw, rows_pw)], dst_all)
        _fill_const(zbuf, STRIPE, 16, 0.0)
        _fill_const(ones_v, CHUNK, 16, 1.0)
        _zero_shared(zbuf, deg_sh, s, n_nodes)
        _zero_shared(zbuf, cnt_sh, s, n_pool)
        plsc.subcore_barrier()

        def ebody(i, carry):
            pltpu.sync_copy(ones_v, deg_sh.at[dst_all.at[i]], add=True)
            return carry
        lax.fori_loop(0, rows_pw, ebody, 0)

        def pbody(i, carry):
            r = w + i * N_WORKERS

            @pl.when(r < pool_rows)
            def _():
                pltpu.sync_copy(pool_hbm.at[r], idx_v)
                pltpu.sync_copy(ones_v, cnt_sh.at[idx_v], add=True)
            return carry
        lax.fori_loop(0, pool_iters, pbody, 0)

        plsc.subcore_barrier()
        _copy_out(deg_sh, deg_hbm, c, s, n_nodes)
        _copy_out(cnt_sh, cnt_hbm, c, s, n_pool)

    return counts(edge3d, pool2d)


def _sc_message(msgs, edge3d, n_nodes, n_edges, feat):
    per_worker = n_edges // N_WORKERS
    rows_pw = per_worker // CHUNK
    nbuf = 5
    n_q = rows_pw // nbuf
    assert rows_pw % nbuf == 0
    mesh = plsc.VectorSubcoreMesh(core_axis_name="c", subcore_axis_name="s")

    @functools.partial(
        pl.kernel,
        out_type=jax.ShapeDtypeStruct((N_CORES, n_nodes, feat), BF16),
        mesh=mesh,
        compiler_params=pltpu.CompilerParams(use_tc_tiling_on_sc=False),
        scratch_types=[
            pltpu.VMEM((rows_pw, CHUNK), jnp.int32),
            pltpu.VMEM((rows_pw, CHUNK), jnp.int32),
            [pltpu.VMEM((CHUNK, feat), BF16)] * nbuf,
            pltpu.VMEM_SHARED((n_nodes, feat), BF16),
            [pltpu.SemaphoreType.DMA] * nbuf,
        ],
    )
    def msg(msgs_hbm, edge_hbm, out_hbm, src_all, dst_all, rows_bufs,
            acc_sh, sems):
        c = lax.axis_index("c")
        s = lax.axis_index("s")
        w = s * N_CORES + c

        pltpu.sync_copy(edge_hbm.at[0, pl.ds(w * rows_pw, rows_pw)], src_all)
        pltpu.sync_copy(edge_hbm.at[1, pl.ds(w * rows_pw, rows_pw)], dst_all)
        _fill_const(rows_bufs[0], CHUNK, feat, 0.0, dtype=BF16)
        _zero_shared(rows_bufs[0], acc_sh, s, n_nodes, stripe=CHUNK)
        plsc.subcore_barrier()

        def body(j, carry):
            i0 = j * nbuf
            descs = [
                pltpu.async_copy(msgs_hbm.at[src_all.at[i0 + k]],
                                 rows_bufs[k], sems[k])
                for k in range(nbuf)
            ]
            for k in range(nbuf):
                descs[k].wait()
                pltpu.sync_copy(rows_bufs[k], acc_sh.at[dst_all.at[i0 + k]],
                                add=True)
            return carry
        lax.fori_loop(0, n_q, body, 0)

        plsc.subcore_barrier()
        _copy_out(acc_sh, out_hbm, c, s, n_nodes)

    return msg(msgs, edge3d)


_BLK = 2000


def _tc_prep0(x, W0, degp):
    n, d_in = x.shape
    d_out = W0.shape[1]

    def body(x_ref, w_ref, degp_ref, m_ref, dis_ref):
        deg = degp_ref[0, :, 0] + degp_ref[1, :, 0]
        dis = jnp.where(deg > 0.0, lax.rsqrt(jnp.maximum(deg, 1.0)), 0.0)
        h = jnp.dot(x_ref[...], w_ref[...], preferred_element_type=F32)
        m_ref[...] = (h * dis[:, None]).astype(BF16)
        dis_ref[...] = dis[:, None]

    return pl.pallas_call(
        body,
        grid=(n // _BLK,),
        in_specs=[
            pl.BlockSpec((_BLK, d_in), lambda i: (i, 0)),
            pl.BlockSpec((d_in, d_out), lambda i: (0, 0)),
            pl.BlockSpec((2, _BLK, 16), lambda i: (0, i, 0)),
        ],
        out_specs=[
            pl.BlockSpec((_BLK, d_out), lambda i: (i, 0)),
            pl.BlockSpec((_BLK, 1), lambda i: (i, 0)),
        ],
        out_shape=[
            jax.ShapeDtypeStruct((n, d_out), BF16),
            jax.ShapeDtypeStruct((n, 1), F32),
        ],
    )(x, W0, degp)


def _tc_prep_mid(partials, dis, b_prev, W):
    n = partials.shape[1]
    f_prev = partials.shape[2]
    f_out = W.shape[1]
    b2d = b_prev.reshape(1, f_prev)

    def body(p_ref, dis_ref, b_ref, w_ref, m_ref):
        ssum = p_ref[0].astype(F32) + p_ref[1].astype(F32)
        a = jnp.maximum(ssum * dis_ref[...] + b_ref[...], 0.0)
        m = jnp.dot(a, w_ref[...], preferred_element_type=F32) * dis_ref[...]
        m_ref[...] = m.astype(BF16)

    return pl.pallas_call(
        body,
        grid=(n // _BLK,),
        in_specs=[
            pl.BlockSpec((2, _BLK, f_prev), lambda i: (0, i, 0)),
            pl.BlockSpec((_BLK, 1), lambda i: (i, 0)),
            pl.BlockSpec((1, f_prev), lambda i: (0, 0)),
            pl.BlockSpec((f_prev, f_out), lambda i: (0, 0)),
        ],
        out_specs=pl.BlockSpec((_BLK, f_out), lambda i: (i, 0)),
        out_shape=jax.ShapeDtypeStruct((n, f_out), BF16),
    )(partials, dis, b2d, W)


def _tc_tail(partials, dis, b2, pool2d_blk, cntp, x_pool1, W_lin, b_lin):
    n = partials.shape[1]
    f = partials.shape[2]
    n_pool = x_pool1.shape[0]
    n_classes = W_lin.shape[1]
    b2d = b2.reshape(1, f)
    bl2d = b_lin.reshape(1, n_classes)
    n_blk = n // _BLK

    def body(p_ref, dis_ref, b_ref, pool_ref, c_ref, xp_ref, wa_ref, wb_ref,
             bl_ref, o_ref, acc_ref):
        i = pl.program_id(0)
        x_mid = ((p_ref[0].astype(F32) + p_ref[1].astype(F32))
                 * dis_ref[...] + b_ref[...])
        ids = pool_ref[0, 0, :]
        seg = lax.broadcasted_iota(jnp.int32, (n_pool, _BLK), 0)
        onehot = (seg == ids[None, :]).astype(F32)
        contrib = jnp.dot(onehot, x_mid, preferred_element_type=F32)

        @pl.when(i == 0)
        def _():
            acc_ref[...] = contrib

        @pl.when(i > 0)
        def _():
            acc_ref[...] += contrib

        @pl.when(i == n_blk - 1)
        def _():
            cnt = c_ref[0, :, 0] + c_ref[1, :, 0]
            x_pre = acc_ref[...] / jnp.maximum(cnt, 1.0)[:, None]
            m_a = jnp.sum(x_pre, axis=0, keepdims=True) / n_pool
            m_b = jnp.sum(xp_ref[...], axis=0, keepdims=True) / n_pool
            logits = (jnp.dot(m_a, wa_ref[...], preferred_element_type=F32)
                      + jnp.dot(m_b, wb_ref[...], preferred_element_type=F32)
                      + bl_ref[...])
            e = jnp.exp(logits - jnp.max(logits, axis=1, keepdims=True))
            o_ref[...] = e / jnp.sum(e, axis=1, keepdims=True)

    return pl.pallas_call(
        body,
        grid=(n_blk,),
        in_specs=[
            pl.BlockSpec((2, _BLK, f), lambda i: (0, i, 0)),
            pl.BlockSpec((_BLK, 1), lambda i: (i, 0)),
            pl.BlockSpec((1, f), lambda i: (0, 0)),
            pl.BlockSpec((1, 1, _BLK), lambda i: (i, 0, 0)),
            pl.BlockSpec((2, n_pool, 16), lambda i: (0, 0, 0)),
            pl.BlockSpec((n_pool, 8), lambda i: (0, 0)),
            pl.BlockSpec((f, n_classes), lambda i: (0, 0)),
            pl.BlockSpec((8, n_classes), lambda i: (0, 0)),
            pl.BlockSpec((1, n_classes), lambda i: (0, 0)),
        ],
        out_specs=pl.BlockSpec((1, n_classes), lambda i: (0, 0)),
        out_shape=jax.ShapeDtypeStruct((1, n_classes), F32),
        scratch_shapes=[pltpu.VMEM((n_pool, f), F32)],
    )(partials, dis, b2d, pool2d_blk, cntp, x_pool1, W_lin[:f], W_lin[f:],
      bl2d)


def kernel(x, edge_index, batch, pool1, x_pool1, W0, b0, W1, b1, W2, b2,
           W_lin, b_lin):
    n_nodes = x.shape[0]
    n_edges = edge_index.shape[1]
    n_pool = x_pool1.shape[0]

    edge3d = edge_index.reshape(2, n_edges // CHUNK, CHUNK)
    pool2d = pool1.reshape(n_nodes // CHUNK, CHUNK)
    degp, cntp = _sc_counts(edge3d, pool2d, n_nodes, n_edges, n_pool)

    m0, dis = _tc_prep0(x, W0, degp)
    p0 = _sc_message(m0, edge3d, n_nodes, n_edges, W0.shape[1])
    m1 = _tc_prep_mid(p0, dis, b0, W1)
    p1 = _sc_message(m1, edge3d, n_nodes, n_edges, W1.shape[1])
    m2 = _tc_prep_mid(p1, dis, b1, W2)
    p2 = _sc_message(m2, edge3d, n_nodes, n_edges, W2.shape[1])

    pool3d_blk = pool1.reshape(n_nodes // _BLK, 1, _BLK)
    return _tc_tail(p2, dis, b2, pool3d_blk, cntp, x_pool1, W_lin, b_lin)
